# Initial kernel scaffold; baseline (speedup 1.0000x reference)
import jax
import jax.numpy as jnp
from jax.experimental import pallas as pl


def kernel(queries, items):
    raise NotImplementedError("write your pallas kernel here")


# speedup vs baseline: 186.8568x; 186.8568x over previous
import functools

import jax
import jax.numpy as jnp
from jax import lax
from jax.experimental import pallas as pl
from jax.experimental.pallas import tpu as pltpu
from jax.experimental.pallas import tpu_sc as plsc

NQ = 1024
D = 32
NI = 100000
BLK = 2048
NBLK = (NI + BLK - 1) // BLK

_NEG_INF = float("-inf")
_BIG_I32 = 2**30


def _topk_body(qT_ref, items_ref, out_ref, v1, a1, v2, a2):
    b = pl.program_id(0)

    @pl.when(b == 0)
    def _init():
        v1[...] = jnp.full((1, NQ), _NEG_INF, jnp.float32)
        v2[...] = jnp.full((1, NQ), _NEG_INF, jnp.float32)
        a1[...] = jnp.full((1, NQ), _BIG_I32, jnp.int32)
        a2[...] = jnp.full((1, NQ), _BIG_I32, jnp.int32)

    blk = items_ref[...]
    scores = lax.dot_general(
        blk, qT_ref[...], (((1,), (0,)), ((), ())),
        preferred_element_type=jnp.float32)

    ridx = lax.broadcasted_iota(jnp.int32, (BLK, NQ), 0) + b * BLK
    scores = jnp.where(ridx < NI, scores, _NEG_INF)

    m1 = jnp.max(scores, axis=0, keepdims=True)
    am1 = jnp.min(jnp.where(scores == m1, ridx, _BIG_I32), axis=0,
                  keepdims=True)
    scores2 = jnp.where(ridx == am1, _NEG_INF, scores)
    m2 = jnp.max(scores2, axis=0, keepdims=True)
    am2 = jnp.min(jnp.where(scores2 == m2, ridx, _BIG_I32), axis=0,
                  keepdims=True)

    v1o, a1o, v2o, a2o = v1[...], a1[...], v2[...], a2[...]
    c = m1 > v1o
    v1[...] = jnp.where(c, m1, v1o)
    a1[...] = jnp.where(c, am1, a1o)
    t = m2 > v1o
    u = m1 > v2o
    v2[...] = jnp.where(c, jnp.where(t, m2, v1o), jnp.where(u, m1, v2o))
    a2[...] = jnp.where(c, jnp.where(t, am2, a1o), jnp.where(u, am1, a2o))

    @pl.when(b == NBLK - 1)
    def _emit():
        out_ref[0:1, :] = a1[...]
        out_ref[1:2, :] = a2[...]


def _topk_call(qT, items):
    return pl.pallas_call(
        _topk_body,
        grid=(NBLK,),
        in_specs=[
            pl.BlockSpec((D, NQ), lambda b: (0, 0)),
            pl.BlockSpec((BLK, D), lambda b: (b, 0)),
        ],
        out_specs=pl.BlockSpec((2, NQ), lambda b: (0, 0)),
        out_shape=jax.ShapeDtypeStruct((2, NQ), jnp.int32),
        scratch_shapes=[
            pltpu.VMEM((1, NQ), jnp.float32),
            pltpu.VMEM((1, NQ), jnp.int32),
            pltpu.VMEM((1, NQ), jnp.float32),
            pltpu.VMEM((1, NQ), jnp.int32),
        ],
    )(qT, items)


def _sc_gather(items, idx):
    info = plsc.get_sparse_core_info()
    nw = info.num_cores * info.num_subcores
    nb = 2 * NQ
    b_per_w = nb // nw
    mesh = plsc.VectorSubcoreMesh(core_axis_name="c", subcore_axis_name="s")

    @functools.partial(
        pl.kernel,
        out_type=jax.ShapeDtypeStruct((nb, D), jnp.float32),
        mesh=mesh,
        scratch_types=[
            pltpu.VMEM((b_per_w,), jnp.int32),
            pltpu.VMEM((b_per_w, D), jnp.float32),
            pltpu.SemaphoreType.DMA,
        ],
        compiler_params=pltpu.CompilerParams(use_tc_tiling_on_sc=False),
    )
    def gather_kernel(table_hbm, idx_hbm, out_hbm, idx_v, rows_v, sem):
        wid = lax.axis_index("s") * info.num_cores + lax.axis_index("c")
        base = wid * b_per_w
        pltpu.sync_copy(idx_hbm.at[pl.ds(base, b_per_w)], idx_v)
        pltpu.async_copy(table_hbm.at[idx_v], rows_v, sem).wait()
        pltpu.sync_copy(rows_v, out_hbm.at[pl.ds(base, b_per_w)])

    return gather_kernel(items, idx)


def _loss_body(q_ref, pos_ref, neg_ref, out_ref):
    q = q_ref[...]
    eps = 1e-6
    dp = jnp.sqrt(jnp.sum((q - pos_ref[...] + eps) ** 2, axis=1,
                          keepdims=True))
    dn = jnp.sqrt(jnp.sum((q - neg_ref[...] + eps) ** 2, axis=1,
                          keepdims=True))
    losses = jnp.maximum(dp - dn + 1.0, 0.0)
    out_ref[0, 0] = jnp.sum(losses) * (1.0 / NQ)


def _loss_call(q, pos, neg):
    return pl.pallas_call(
        _loss_body,
        in_specs=[
            pl.BlockSpec((NQ, D), lambda: (0, 0)),
            pl.BlockSpec((NQ, D), lambda: (0, 0)),
            pl.BlockSpec((NQ, D), lambda: (0, 0)),
        ],
        out_specs=pl.BlockSpec(memory_space=pltpu.SMEM),
        out_shape=jax.ShapeDtypeStruct((1, 1), jnp.float32),
    )(q, pos, neg)


def kernel(queries, items):
    q = queries.reshape(NQ, D)
    qT = q.T
    idx2 = _topk_call(qT, items)
    gathered = _sc_gather(items, idx2.reshape(2 * NQ))
    pos = gathered[:NQ]
    neg = gathered[NQ:]
    return _loss_call(q, pos, neg).reshape(())


# speedup vs baseline: 190.3934x; 1.0189x over previous
import functools

import jax
import jax.numpy as jnp
from jax import lax
from jax.experimental import pallas as pl
from jax.experimental.pallas import tpu as pltpu
from jax.experimental.pallas import tpu_sc as plsc

NQ = 1024
D = 32
NI = 100000
BLK = 2000
NBLK = NI // BLK

_NEG_INF = float("-inf")
_BIG_I32 = 2**30


def _topk_body(q_ref, items_ref, out_ref, sc_ref, m1s, m2s, a1s, a2s):
    b = pl.program_id(0)
    nstrip = BLK // 8

    sc_ref[...] = lax.dot_general(
        items_ref[...], q_ref[...], (((1,), (1,)), ((), ())),
        preferred_element_type=jnp.float32)

    @pl.when(b == 0)
    def _init():
        m1s[...] = jnp.full((8, NQ), _NEG_INF, jnp.float32)
        m2s[...] = jnp.full((8, NQ), _NEG_INF, jnp.float32)
        a1s[...] = jnp.zeros((8, NQ), jnp.int32)
        a2s[...] = jnp.zeros((8, NQ), jnp.int32)

    def step(t, carry):
        M1, M2, A1, A2 = carry
        s = sc_ref[pl.ds(t * 8, 8), :]
        k = jnp.full((8, NQ), b * nstrip + t, jnp.int32)
        c1 = s > M1
        c2 = s > M2
        M2n = jnp.where(c1, M1, jnp.where(c2, s, M2))
        A2n = jnp.where(c1, A1, jnp.where(c2, k, A2))
        M1n = jnp.where(c1, s, M1)
        A1n = jnp.where(c1, k, A1)
        return M1n, M2n, A1n, A2n

    M1, M2, A1, A2 = lax.fori_loop(
        0, nstrip, step, (m1s[...], m2s[...], a1s[...], a2s[...]),
        unroll=2)
    m1s[...], m2s[...], a1s[...], a2s[...] = M1, M2, A1, A2

    @pl.when(b == NBLK - 1)
    def _emit():
        sub = lax.broadcasted_iota(jnp.int32, (8, NQ), 0)
        rows1 = A1 * 8 + sub
        rows2 = A2 * 8 + sub
        g1 = jnp.max(M1, axis=0, keepdims=True)
        am1 = jnp.min(jnp.where(M1 == g1, rows1, _BIG_I32), axis=0,
                      keepdims=True)
        winner = rows1 == am1
        v2 = jnp.where(winner, M2, M1)
        i2 = jnp.where(winner, rows2, rows1)
        g2 = jnp.max(v2, axis=0, keepdims=True)
        am2 = jnp.min(jnp.where(v2 == g2, i2, _BIG_I32), axis=0,
                      keepdims=True)
        out_ref[0:1, :] = am1
        out_ref[1:2, :] = am2


def _topk_call(q, items):
    return pl.pallas_call(
        _topk_body,
        grid=(NBLK,),
        in_specs=[
            pl.BlockSpec((NQ, D), lambda b: (0, 0)),
            pl.BlockSpec((BLK, D), lambda b: (b, 0)),
        ],
        out_specs=pl.BlockSpec((2, NQ), lambda b: (0, 0)),
        out_shape=jax.ShapeDtypeStruct((2, NQ), jnp.int32),
        scratch_shapes=[
            pltpu.VMEM((BLK, NQ), jnp.float32),
            pltpu.VMEM((8, NQ), jnp.float32),
            pltpu.VMEM((8, NQ), jnp.float32),
            pltpu.VMEM((8, NQ), jnp.int32),
            pltpu.VMEM((8, NQ), jnp.int32),
        ],
    )(q, items)


def _sc_gather(items, idx):
    info = plsc.get_sparse_core_info()
    nw = info.num_cores * info.num_subcores
    nb = 2 * NQ
    b_per_w = nb // nw
    mesh = plsc.VectorSubcoreMesh(core_axis_name="c", subcore_axis_name="s")

    @functools.partial(
        pl.kernel,
        out_type=jax.ShapeDtypeStruct((nb, D), jnp.float32),
        mesh=mesh,
        scratch_types=[
            pltpu.VMEM((b_per_w,), jnp.int32),
            pltpu.VMEM((b_per_w, D), jnp.float32),
            pltpu.SemaphoreType.DMA,
        ],
        compiler_params=pltpu.CompilerParams(use_tc_tiling_on_sc=False),
    )
    def gather_kernel(table_hbm, idx_hbm, out_hbm, idx_v, rows_v, sem):
        wid = lax.axis_index("s") * info.num_cores + lax.axis_index("c")
        base = wid * b_per_w
        pltpu.sync_copy(idx_hbm.at[pl.ds(base, b_per_w)], idx_v)
        pltpu.async_copy(table_hbm.at[idx_v], rows_v, sem).wait()
        pltpu.sync_copy(rows_v, out_hbm.at[pl.ds(base, b_per_w)])

    return gather_kernel(items, idx)


def _loss_body(q_ref, pos_ref, neg_ref, out_ref):
    q = q_ref[...]
    eps = 1e-6
    dp = jnp.sqrt(jnp.sum((q - pos_ref[...] + eps) ** 2, axis=1,
                          keepdims=True))
    dn = jnp.sqrt(jnp.sum((q - neg_ref[...] + eps) ** 2, axis=1,
                          keepdims=True))
    losses = jnp.maximum(dp - dn + 1.0, 0.0)
    out_ref[0, 0] = jnp.sum(losses) * (1.0 / NQ)


def _loss_call(q, gathered):
    return pl.pallas_call(
        _loss_body,
        grid=(1,),
        in_specs=[
            pl.BlockSpec((NQ, D), lambda i: (0, 0)),
            pl.BlockSpec((NQ, D), lambda i: (0, 0)),
            pl.BlockSpec((NQ, D), lambda i: (1, 0)),
        ],
        out_specs=pl.BlockSpec(memory_space=pltpu.SMEM),
        out_shape=jax.ShapeDtypeStruct((1, 1), jnp.float32),
    )(q, gathered, gathered)


def kernel(queries, items):
    q = queries.reshape(NQ, D)
    idx2 = _topk_call(q, items)
    gathered = _sc_gather(items, idx2.reshape(2 * NQ))
    return _loss_call(q, gathered).reshape(())


# speedup vs baseline: 227.4809x; 1.1948x over previous
import functools

import jax
import jax.numpy as jnp
from jax import lax
from jax.experimental import pallas as pl
from jax.experimental.pallas import tpu as pltpu
from jax.experimental.pallas import tpu_sc as plsc

NQ = 1024
D = 32
NI = 100000
BLK = 2000
NBLK = NI // BLK

_NEG_INF = float("-inf")
_BIG_I32 = 2**30


def _topk_body(q_ref, items_ref, out_ref, sc_ref, m1s, m2s, a1s, a2s):
    b = pl.program_id(0)
    nstrip = BLK // 8

    sc_ref[...] = lax.dot_general(
        items_ref[...], q_ref[...], (((1,), (1,)), ((), ())),
        preferred_element_type=jnp.float32)

    @pl.when(b == 0)
    def _init():
        m1s[...] = jnp.full((8, NQ), _NEG_INF, jnp.float32)
        m2s[...] = jnp.full((8, NQ), _NEG_INF, jnp.float32)
        a1s[...] = jnp.zeros((8, NQ), jnp.int32)
        a2s[...] = jnp.zeros((8, NQ), jnp.int32)

    def step(t, carry):
        M1, M2, A1, A2 = carry
        s = sc_ref[pl.ds(t * 8, 8), :]
        k = jnp.full((8, NQ), b * nstrip + t, jnp.int32)
        c1 = s > M1
        c2 = s > M2
        M2n = jnp.where(c1, M1, jnp.where(c2, s, M2))
        A2n = jnp.where(c1, A1, jnp.where(c2, k, A2))
        M1n = jnp.where(c1, s, M1)
        A1n = jnp.where(c1, k, A1)
        return M1n, M2n, A1n, A2n

    M1, M2, A1, A2 = lax.fori_loop(
        0, nstrip, step, (m1s[...], m2s[...], a1s[...], a2s[...]),
        unroll=25)
    m1s[...], m2s[...], a1s[...], a2s[...] = M1, M2, A1, A2

    @pl.when(b == NBLK - 1)
    def _emit():
        sub = lax.broadcasted_iota(jnp.int32, (8, NQ), 0)
        rows1 = A1 * 8 + sub
        rows2 = A2 * 8 + sub
        g1 = jnp.max(M1, axis=0, keepdims=True)
        am1 = jnp.min(jnp.where(M1 == g1, rows1, _BIG_I32), axis=0,
                      keepdims=True)
        winner = rows1 == am1
        v2 = jnp.where(winner, M2, M1)
        i2 = jnp.where(winner, rows2, rows1)
        g2 = jnp.max(v2, axis=0, keepdims=True)
        am2 = jnp.min(jnp.where(v2 == g2, i2, _BIG_I32), axis=0,
                      keepdims=True)
        out_ref[0:1, :] = am1
        out_ref[1:2, :] = am2


def _topk_call(q, items):
    return pl.pallas_call(
        _topk_body,
        grid=(NBLK,),
        in_specs=[
            pl.BlockSpec((NQ, D), lambda b: (0, 0)),
            pl.BlockSpec((BLK, D), lambda b: (b, 0)),
        ],
        out_specs=pl.BlockSpec((2, NQ), lambda b: (0, 0)),
        out_shape=jax.ShapeDtypeStruct((2, NQ), jnp.int32),
        scratch_shapes=[
            pltpu.VMEM((BLK, NQ), jnp.float32),
            pltpu.VMEM((8, NQ), jnp.float32),
            pltpu.VMEM((8, NQ), jnp.float32),
            pltpu.VMEM((8, NQ), jnp.int32),
            pltpu.VMEM((8, NQ), jnp.int32),
        ],
    )(q, items)


def _sc_gather(items, idx):
    info = plsc.get_sparse_core_info()
    nw = info.num_cores * info.num_subcores
    nb = 2 * NQ
    b_per_w = nb // nw
    mesh = plsc.VectorSubcoreMesh(core_axis_name="c", subcore_axis_name="s")

    @functools.partial(
        pl.kernel,
        out_type=jax.ShapeDtypeStruct((nb, D), jnp.float32),
        mesh=mesh,
        scratch_types=[
            pltpu.VMEM((b_per_w,), jnp.int32),
            pltpu.VMEM((b_per_w, D), jnp.float32),
            pltpu.SemaphoreType.DMA,
        ],
        compiler_params=pltpu.CompilerParams(use_tc_tiling_on_sc=False),
    )
    def gather_kernel(table_hbm, idx_hbm, out_hbm, idx_v, rows_v, sem):
        wid = lax.axis_index("s") * info.num_cores + lax.axis_index("c")
        base = wid * b_per_w
        pltpu.sync_copy(idx_hbm.at[pl.ds(base, b_per_w)], idx_v)
        pltpu.async_copy(table_hbm.at[idx_v], rows_v, sem).wait()
        pltpu.sync_copy(rows_v, out_hbm.at[pl.ds(base, b_per_w)])

    return gather_kernel(items, idx)


def _loss_body(q_ref, pos_ref, neg_ref, out_ref):
    q = q_ref[...]
    eps = 1e-6
    dp = jnp.sqrt(jnp.sum((q - pos_ref[...] + eps) ** 2, axis=1,
                          keepdims=True))
    dn = jnp.sqrt(jnp.sum((q - neg_ref[...] + eps) ** 2, axis=1,
                          keepdims=True))
    losses = jnp.maximum(dp - dn + 1.0, 0.0)
    out_ref[0, 0] = jnp.sum(losses) * (1.0 / NQ)


def _loss_call(q, gathered):
    return pl.pallas_call(
        _loss_body,
        grid=(1,),
        in_specs=[
            pl.BlockSpec((NQ, D), lambda i: (0, 0)),
            pl.BlockSpec((NQ, D), lambda i: (0, 0)),
            pl.BlockSpec((NQ, D), lambda i: (1, 0)),
        ],
        out_specs=pl.BlockSpec(memory_space=pltpu.SMEM),
        out_shape=jax.ShapeDtypeStruct((1, 1), jnp.float32),
    )(q, gathered, gathered)


def kernel(queries, items):
    q = queries.reshape(NQ, D)
    idx2 = _topk_call(q, items)
    gathered = _sc_gather(items, idx2.reshape(2 * NQ))
    return _loss_call(q, gathered).reshape(())
